# baseline (speedup 1.0000x reference)
import numpy as np
import jax
import jax.numpy as jnp
from jax import lax
from jax.experimental import pallas as pl
from jax.experimental.pallas import tpu as pltpu

B = 64
P = 32768
C = 21
IB = 8
NT = 1
GT = 256
P8 = P // 8
THR = 0.3
F32 = jnp.float32

_s = np.arange(C)[:, None, None]
_l = np.arange(128)[None, :, None]
_p = np.arange(128)[None, None, :]
_f = 128 * _s + _l
W_SSE = (_f // C == _p).astype(np.float32)
W_S0 = (_f == C * _p).astype(np.float32)
D_IDX = (_f - C * _p).astype(np.int32)
D_CMB = np.concatenate([np.where(W_S0 > 0, 10 ** 6, -1).astype(np.int32), D_IDX],
                       axis=2)
_s6 = np.arange(6)[:, None, None]
_f6 = 128 * _s6 + _l
_j6 = np.arange(6 * 128)[None, None, :] // 128
_p6 = np.arange(6 * 128)[None, None, :] % 128
W_LOC = (_f6 == 6 * _p6 + _j6).astype(np.float32)

_BN = (((2,), (1,)), ((1,), (0,)))


def _iou_terms(cx, cy, w, h, px, py, pw, ph):
    bx1 = cx - w * 0.5
    by1 = cy - h * 0.5
    bx2 = cx + w * 0.5
    by2 = cy + h * 0.5
    px1 = px - pw * 0.5
    py1 = py - ph * 0.5
    px2 = px + pw * 0.5
    py2 = py + ph * 0.5
    wx = jnp.maximum(jnp.minimum(bx2, px2) - jnp.maximum(bx1, px1), 0.0)
    wy = jnp.maximum(jnp.minimum(by2, py2) - jnp.maximum(by1, py1), 0.0)
    inter = wx * wy
    return inter / (w * h + pw * ph - inter + 1e-10)


def _best_kernel(tgt_ref, pr_ref, best_ref):
    cx = tgt_ref[0, 0, 0]
    cy = tgt_ref[0, 0, 1]
    w = tgt_ref[0, 0, 2]
    h = tgt_ref[0, 0, 3]
    px, py, pw, ph = pr_ref[0], pr_ref[1], pr_ref[2], pr_ref[3]
    iou = _iou_terms(cx, cy, w, h, px, py, pw, ph)
    m = jnp.max(iou)
    r_i = lax.broadcasted_iota(jnp.int32, iou.shape, 0).astype(F32)
    c_i = lax.broadcasted_iota(jnp.int32, iou.shape, 1).astype(F32)
    gp = r_i * float(P8) + c_i
    best = jnp.min(jnp.where(iou == m, gp, F32(P)))
    best_ref[...] = jnp.full((1, 1, 128), best, F32)


def _main_kernel(tgt_ref, best_ref, pr_ref, sc_ref, lc_ref, wsse_ref,
                 didx_ref, wloc_ref, acc_ref, tot_ref,
                 wslab_ref, ce_ref):
    n = pl.program_id(1)
    b = pl.program_id(0)
    cx = tgt_ref[0, 0, 0]
    cy = tgt_ref[0, 0, 1]
    w = tgt_ref[0, 0, 2]
    h = tgt_ref[0, 0, 3]
    sn = tgt_ref[0, 0, 5]
    cs = tgt_ref[0, 0, 6]
    lab_i = lax.convert_element_type(tgt_ref[0, 0, 7], jnp.int32)
    best = best_ref[0, 0, 0]
    px, py, pw, ph = (pr_ref[0, 0], pr_ref[1, 0], pr_ref[2, 0], pr_ref[3, 0])

    @pl.when(n == 0)
    def _():
        wslab_ref[...] = ((didx_ref[...] == 10 ** 6)
                          | (didx_ref[...] == lab_i)).astype(jnp.bfloat16)

    iou = _iou_terms(cx, cy, w, h, px, py, pw, ph)
    g_i = lax.broadcasted_iota(jnp.int32, iou.shape, 0)
    l_i = lax.broadcasted_iota(jnp.int32, iou.shape, 1)
    gp = ((n * (GT * 128) + g_i * 128 + l_i)).astype(F32)
    pos = (iou >= THR) | (gp == best)
    posf = pos.astype(F32)
    npos_t = jnp.sum(posf)

    wsse_b = wsse_ref[...]
    wcmb_b = wslab_ref[...]
    wloc_b = wloc_ref[...]
    conf_t = loc_t = ang_t = jnp.float32(0.0)
    CH = 32
    for g in range(GT // CH):
        sl = slice(g * CH, (g + 1) * CH)
        x = sc_ref[0, sl].astype(jnp.bfloat16)
        e = jnp.exp(sc_ref[0, sl]).astype(jnp.bfloat16)
        sse = jnp.sum(lax.dot_general(e, wsse_b, _BN,
                                      preferred_element_type=F32), axis=0)
        both = jnp.sum(lax.dot_general(x, wcmb_b, _BN,
                                       preferred_element_type=F32), axis=0)
        s0 = both[:, 0:128]
        slab = both[:, 128:256]
        lse = jnp.log(sse)
        ce0 = lse - s0
        posc = pos[sl]
        pfc = posf[sl]
        conf_t += jnp.sum(pfc * (lse - slab))
        ce_ref[pl.ds(n * GT + g * CH, CH), :] = jnp.where(posc, 0.0, ce0)

        xl = lc_ref[0, sl].astype(jnp.bfloat16)
        lt = jnp.sum(lax.dot_general(xl, wloc_b, _BN,
                                     preferred_element_type=F32), axis=0)
        pxc, pyc, pwc, phc = px[sl], py[sl], pw[sl], ph[sl]
        d0 = lt[:, 0:128] - (cx - pxc) / (pwc * 0.1)
        d1 = lt[:, 128:256] - (cy - pyc) / (phc * 0.1)
        d2 = lt[:, 256:384] - 5.0 * jnp.log(w / pwc)
        d3 = lt[:, 384:512] - 5.0 * jnp.log(h / phc)
        d4 = lt[:, 512:640] - sn
        d5 = lt[:, 640:768] - cs
        for d in (d0, d1, d2, d3):
            ad = jnp.abs(d)
            loc_t += jnp.sum(pfc * jnp.where(ad < 1.0, 0.5 * d * d, ad - 0.5))
        ang_t += jnp.sum(pfc * (d4 * d4 + d5 * d5))

    lane = lax.broadcasted_iota(jnp.int32, (1, 1, 128), 2)
    vec = (jnp.where(lane == 0, npos_t, 0.0)
           + jnp.where(lane == 1, conf_t, 0.0)
           + jnp.where(lane == 2, loc_t, 0.0)
           + jnp.where(lane == 3, ang_t, 0.0))

    @pl.when(n == 0)
    def _():
        acc_ref[...] = vec

    @pl.when(n != 0)
    def _():
        acc_ref[...] = acc_ref[...] + vec

    @pl.when((b == 0) & (n == 0))
    def _():
        tot_ref[...] = jnp.zeros((1, 1, 128), F32)

    @pl.when(n == NT - 1)
    def _():
        x = ce_ref[...]
        bits = lax.bitcast_convert_type(x, jnp.int32)
        npos = acc_ref[0, 0, 0]
        kf = jnp.minimum(npos * 3.0, F32(P))

        def body(_, lohi):
            lo, hi = lohi
            mid = lo + lax.div(hi - lo, jnp.int32(2))
            cnt = jnp.sum((bits > mid).astype(F32))
            take = cnt >= kf
            return (jnp.where(take, mid, lo), jnp.where(take, hi, mid))

        _, hi = lax.fori_loop(0, 31, body,
                              (jnp.int32(-1), jnp.int32(0x7F800000)))
        vkf = lax.bitcast_convert_type(hi, F32)
        gtm = bits > hi
        sum_gt = jnp.sum(jnp.where(gtm, x, 0.0))
        cnt_gt = jnp.sum(gtm.astype(F32))
        topk = sum_gt + (kf - cnt_gt) * vkf
        vec2 = (jnp.where(lane == 0, npos, 0.0)
                + jnp.where(lane == 1, acc_ref[0, 0, 1] + topk, 0.0)
                + jnp.where(lane == 2, acc_ref[0, 0, 2], 0.0)
                + jnp.where(lane == 3, acc_ref[0, 0, 3], 0.0))
        tot_ref[...] = tot_ref[...] + vec2


def kernel(predicted_locs, predicted_scores, target, priors_cxcy):
    prt = priors_cxcy.T
    priors_b = prt.reshape(4, 8, P8)
    priors_m = prt.reshape(4, NT, GT, 128)
    scf = predicted_scores.reshape(B, NT * GT, C, 128)
    lcf = predicted_locs.reshape(B, NT * GT, 6, 128)
    wsse = jnp.asarray(W_SSE).astype(jnp.bfloat16)
    didx = jnp.asarray(D_CMB)
    wloc = jnp.asarray(W_LOC).astype(jnp.bfloat16)

    best = pl.pallas_call(
        _best_kernel,
        grid=(B,),
        in_specs=[
            pl.BlockSpec((1, 1, 8), lambda b: (b, 0, 0)),
            pl.BlockSpec((4, 8, P8), lambda b: (0, 0, 0)),
        ],
        out_specs=pl.BlockSpec((1, 1, 128), lambda b: (b, 0, 0)),
        out_shape=jax.ShapeDtypeStruct((B, 1, 128), F32),
    )(target, priors_b)

    acc, tot3 = pl.pallas_call(
        _main_kernel,
        grid=(B, NT),
        in_specs=[
            pl.BlockSpec((1, 1, 8), lambda b, n: (b, 0, 0)),
            pl.BlockSpec((1, 1, 128), lambda b, n: (b, 0, 0)),
            pl.BlockSpec((4, 1, GT, 128), lambda b, n: (0, n, 0, 0)),
            pl.BlockSpec((1, GT, C, 128), lambda b, n: (b, n, 0, 0)),
            pl.BlockSpec((1, GT, 6, 128), lambda b, n: (b, n, 0, 0)),
            pl.BlockSpec((C, 128, 128), lambda b, n: (0, 0, 0)),
            pl.BlockSpec((C, 128, 256), lambda b, n: (0, 0, 0)),
            pl.BlockSpec((6, 128, 768), lambda b, n: (0, 0, 0)),
        ],
        out_specs=[
            pl.BlockSpec((1, 1, 128), lambda b, n: (b, 0, 0)),
            pl.BlockSpec((1, 1, 128), lambda b, n: (0, 0, 0)),
        ],
        out_shape=[
            jax.ShapeDtypeStruct((B, 1, 128), F32),
            jax.ShapeDtypeStruct((1, 1, 128), F32),
        ],
        scratch_shapes=[pltpu.VMEM((C, 128, 256), jnp.bfloat16),
                        pltpu.VMEM((NT * GT, 128), F32)],
    )(target, best, priors_m, scf, lcf, wsse, didx, wloc)

    n = tot3[0, 0, 0]
    conf = tot3[0, 0, 1] / n
    loc = tot3[0, 0, 2] / (n * 4.0)
    ang = 25.0 * tot3[0, 0, 3] / (n * 2.0)
    return (conf, loc, ang, conf + loc + ang)


# speedup vs baseline: 1.2841x; 1.2841x over previous
import numpy as np
import jax
import jax.numpy as jnp
from jax import lax
from jax.experimental import pallas as pl
from jax.experimental.pallas import tpu as pltpu

B = 64
P = 32768
C = 21
IB = 8
NT = 1
GT = 256
P8 = P // 8
THR = 0.3
F32 = jnp.float32

_s = np.arange(C)[:, None, None]
_l = np.arange(128)[None, :, None]
_p = np.arange(128)[None, None, :]
_f = 128 * _s + _l
W_SSE = (_f // C == _p).astype(np.float32)
W_S0 = (_f == C * _p).astype(np.float32)
D_IDX = (_f - C * _p).astype(np.int32)
_s6 = np.arange(6)[:, None, None]
_f6 = 128 * _s6 + _l
_j6 = np.arange(6 * 128)[None, None, :] // 128
_p6 = np.arange(6 * 128)[None, None, :] % 128
W_LOC = (_f6 == 6 * _p6 + _j6).astype(np.float32)

_BN = (((2,), (1,)), ((1,), (0,)))


def _iou_terms(cx, cy, w, h, px, py, pw, ph):
    bx1 = cx - w * 0.5
    by1 = cy - h * 0.5
    bx2 = cx + w * 0.5
    by2 = cy + h * 0.5
    px1 = px - pw * 0.5
    py1 = py - ph * 0.5
    px2 = px + pw * 0.5
    py2 = py + ph * 0.5
    wx = jnp.maximum(jnp.minimum(bx2, px2) - jnp.maximum(bx1, px1), 0.0)
    wy = jnp.maximum(jnp.minimum(by2, py2) - jnp.maximum(by1, py1), 0.0)
    inter = wx * wy
    return inter / (w * h + pw * ph - inter + 1e-10)


def _best_kernel(tgt_ref, pr_ref, best_ref):
    cx = tgt_ref[0, 0, 0]
    cy = tgt_ref[0, 0, 1]
    w = tgt_ref[0, 0, 2]
    h = tgt_ref[0, 0, 3]
    px, py, pw, ph = pr_ref[0], pr_ref[1], pr_ref[2], pr_ref[3]
    iou = _iou_terms(cx, cy, w, h, px, py, pw, ph)
    m = jnp.max(iou)
    r_i = lax.broadcasted_iota(jnp.int32, iou.shape, 0).astype(F32)
    c_i = lax.broadcasted_iota(jnp.int32, iou.shape, 1).astype(F32)
    gp = r_i * float(P8) + c_i
    best = jnp.min(jnp.where(iou == m, gp, F32(P)))
    best_ref[...] = jnp.full((1, 1, 128), best, F32)


def _main_kernel(tgt_ref, best_ref, pr_ref, sc_ref, lc_ref, wsse_ref,
                 ws0_ref, didx_ref, wloc_ref, ce_ref, acc_ref, wslab_ref):
    n = pl.program_id(1)
    b = pl.program_id(0)
    cx = tgt_ref[0, 0, 0]
    cy = tgt_ref[0, 0, 1]
    w = tgt_ref[0, 0, 2]
    h = tgt_ref[0, 0, 3]
    sn = tgt_ref[0, 0, 5]
    cs = tgt_ref[0, 0, 6]
    lab_i = lax.convert_element_type(tgt_ref[0, 0, 7], jnp.int32)
    best = best_ref[0, 0, 0]
    px, py, pw, ph = (pr_ref[0, 0], pr_ref[1, 0], pr_ref[2, 0], pr_ref[3, 0])

    @pl.when(n == 0)
    def _():
        wslab_ref[...] = (didx_ref[...] == lab_i).astype(F32)

    iou = _iou_terms(cx, cy, w, h, px, py, pw, ph)
    g_i = lax.broadcasted_iota(jnp.int32, iou.shape, 0)
    l_i = lax.broadcasted_iota(jnp.int32, iou.shape, 1)
    gp = ((n * (GT * 128) + g_i * 128 + l_i)).astype(F32)
    pos = (iou >= THR) | (gp == best)
    posf = pos.astype(F32)
    npos_t = jnp.sum(posf)

    x = sc_ref[0]
    e = jnp.exp(x)
    sse = jnp.sum(lax.dot_general(e, wsse_ref[...], _BN,
                                  preferred_element_type=F32), axis=0)
    s0 = jnp.sum(lax.dot_general(x, ws0_ref[...], _BN,
                                 preferred_element_type=F32), axis=0)
    slab = jnp.sum(lax.dot_general(x, wslab_ref[...], _BN,
                                   preferred_element_type=F32), axis=0)
    lse = jnp.log(sse)
    ce0 = lse - s0
    conf_t = jnp.sum(posf * (lse - slab))
    ce_ref[0] = jnp.where(pos, 0.0, ce0)

    xl = lc_ref[0]
    lt = jnp.sum(lax.dot_general(xl, wloc_ref[...], _BN,
                                 preferred_element_type=F32), axis=0)
    d0 = lt[:, 0:128] - (cx - px) / (pw * 0.1)
    d1 = lt[:, 128:256] - (cy - py) / (ph * 0.1)
    d2 = lt[:, 256:384] - 5.0 * jnp.log(w / pw)
    d3 = lt[:, 384:512] - 5.0 * jnp.log(h / ph)
    d4 = lt[:, 512:640] - sn
    d5 = lt[:, 640:768] - cs
    loc_t = 0.0
    for d in (d0, d1, d2, d3):
        ad = jnp.abs(d)
        loc_t += jnp.sum(posf * jnp.where(ad < 1.0, 0.5 * d * d, ad - 0.5))
    ang_t = jnp.sum(posf * (d4 * d4 + d5 * d5))

    lane = lax.broadcasted_iota(jnp.int32, (1, 1, 128), 2)
    vec = (jnp.where(lane == 0, npos_t, 0.0)
           + jnp.where(lane == 1, conf_t, 0.0)
           + jnp.where(lane == 2, loc_t, 0.0)
           + jnp.where(lane == 3, ang_t, 0.0))

    @pl.when(n == 0)
    def _():
        acc_ref[...] = vec

    @pl.when(n != 0)
    def _():
        acc_ref[...] = acc_ref[...] + vec


def _mine_kernel(ce_ref, acc_ref, tot_ref):
    g = pl.program_id(0)
    x = ce_ref[...]
    bits = lax.bitcast_convert_type(x, jnp.int32)
    npos = acc_ref[:, :, 0:1]
    kf = jnp.minimum(npos * 3.0, F32(P))

    def body(_, lohi):
        lo, hi = lohi
        mid = lo + lax.div(hi - lo, jnp.int32(2))
        cnt = jnp.sum((bits > mid).astype(F32), axis=(1, 2), keepdims=True)
        take = cnt >= kf
        return (jnp.where(take, mid, lo), jnp.where(take, hi, mid))

    init = (jnp.full((IB, 1, 1), -1, jnp.int32),
            jnp.full((IB, 1, 1), 0x7F800000, jnp.int32))
    _, hi = lax.fori_loop(0, 31, body, init)
    vkf = lax.bitcast_convert_type(hi, F32)
    gtm = bits > hi
    sum_gt = jnp.sum(jnp.where(gtm, x, 0.0), axis=(1, 2), keepdims=True)
    cnt_gt = jnp.sum(gtm.astype(F32), axis=(1, 2), keepdims=True)
    topk = sum_gt + (kf - cnt_gt) * vkf

    lane = lax.broadcasted_iota(jnp.int32, (1, 128), 1)
    vec = (jnp.where(lane == 0, jnp.sum(npos), 0.0)
           + jnp.where(lane == 1, jnp.sum(acc_ref[:, :, 1:2] + topk), 0.0)
           + jnp.where(lane == 2, jnp.sum(acc_ref[:, :, 2:3]), 0.0)
           + jnp.where(lane == 3, jnp.sum(acc_ref[:, :, 3:4]), 0.0))

    @pl.when(g == 0)
    def _():
        tot_ref[...] = vec

    @pl.when(g != 0)
    def _():
        tot_ref[...] = tot_ref[...] + vec


def kernel(predicted_locs, predicted_scores, target, priors_cxcy):
    prt = priors_cxcy.T
    priors_b = prt.reshape(4, 8, P8)
    priors_m = prt.reshape(4, NT, GT, 128)
    scf = predicted_scores.reshape(B, NT * GT, C, 128)
    lcf = predicted_locs.reshape(B, NT * GT, 6, 128)
    wsse = jnp.asarray(W_SSE)
    ws0 = jnp.asarray(W_S0)
    didx = jnp.asarray(D_IDX)
    wloc = jnp.asarray(W_LOC)

    best = pl.pallas_call(
        _best_kernel,
        grid=(B,),
        in_specs=[
            pl.BlockSpec((1, 1, 8), lambda b: (b, 0, 0)),
            pl.BlockSpec((4, 8, P8), lambda b: (0, 0, 0)),
        ],
        out_specs=pl.BlockSpec((1, 1, 128), lambda b: (b, 0, 0)),
        out_shape=jax.ShapeDtypeStruct((B, 1, 128), F32),
    )(target, priors_b)

    ce_neg, acc = pl.pallas_call(
        _main_kernel,
        grid=(B, NT),
        in_specs=[
            pl.BlockSpec((1, 1, 8), lambda b, n: (b, 0, 0)),
            pl.BlockSpec((1, 1, 128), lambda b, n: (b, 0, 0)),
            pl.BlockSpec((4, 1, GT, 128), lambda b, n: (0, n, 0, 0)),
            pl.BlockSpec((1, GT, C, 128), lambda b, n: (b, n, 0, 0)),
            pl.BlockSpec((1, GT, 6, 128), lambda b, n: (b, n, 0, 0)),
            pl.BlockSpec((C, 128, 128), lambda b, n: (0, 0, 0)),
            pl.BlockSpec((C, 128, 128), lambda b, n: (0, 0, 0)),
            pl.BlockSpec((C, 128, 128), lambda b, n: (0, 0, 0)),
            pl.BlockSpec((6, 128, 768), lambda b, n: (0, 0, 0)),
        ],
        out_specs=[
            pl.BlockSpec((1, NT * GT, 128), lambda b, n: (b, 0, 0)),
            pl.BlockSpec((1, 1, 128), lambda b, n: (b, 0, 0)),
        ],
        out_shape=[
            jax.ShapeDtypeStruct((B, NT * GT, 128), F32),
            jax.ShapeDtypeStruct((B, 1, 128), F32),
        ],
        scratch_shapes=[pltpu.VMEM((C, 128, 128), F32)],
    )(target, best, priors_m, scf, lcf, wsse, ws0, didx, wloc)

    tot = pl.pallas_call(
        _mine_kernel,
        grid=(B // IB,),
        in_specs=[
            pl.BlockSpec((IB, NT * GT, 128), lambda g: (g, 0, 0)),
            pl.BlockSpec((IB, 1, 128), lambda g: (g, 0, 0)),
        ],
        out_specs=pl.BlockSpec((1, 128), lambda g: (0, 0)),
        out_shape=jax.ShapeDtypeStruct((1, 128), F32),
    )(ce_neg, acc)

    n = tot[0, 0]
    conf = tot[0, 1] / n
    loc = tot[0, 2] / (n * 4.0)
    ang = 25.0 * tot[0, 3] / (n * 2.0)
    return (conf, loc, ang, conf + loc + ang)


# speedup vs baseline: 1.3002x; 1.0126x over previous
import numpy as np
import jax
import jax.numpy as jnp
from jax import lax
from jax.experimental import pallas as pl
from jax.experimental.pallas import tpu as pltpu

B = 64
P = 32768
C = 21
IB = 8
NT = 1
GT = 256
P8 = P // 8
THR = 0.3
F32 = jnp.float32

_s = np.arange(C)[:, None, None]
_l = np.arange(128)[None, :, None]
_p = np.arange(128)[None, None, :]
_f = 128 * _s + _l
W_SSE = (_f // C == _p).astype(np.float32)
W_S0 = (_f == C * _p).astype(np.float32)
D_IDX = (_f - C * _p).astype(np.int32)
D_CMB = np.concatenate([np.where(W_S0 > 0, 10 ** 6, -1).astype(np.int32),
                        D_IDX], axis=2)
_s6 = np.arange(6)[:, None, None]
_f6 = 128 * _s6 + _l
_j6 = np.arange(6 * 128)[None, None, :] // 128
_p6 = np.arange(6 * 128)[None, None, :] % 128
W_LOC = (_f6 == 6 * _p6 + _j6).astype(np.float32)

_BN = (((2,), (1,)), ((1,), (0,)))


def _iou_terms(cx, cy, w, h, px, py, pw, ph):
    bx1 = cx - w * 0.5
    by1 = cy - h * 0.5
    bx2 = cx + w * 0.5
    by2 = cy + h * 0.5
    px1 = px - pw * 0.5
    py1 = py - ph * 0.5
    px2 = px + pw * 0.5
    py2 = py + ph * 0.5
    wx = jnp.maximum(jnp.minimum(bx2, px2) - jnp.maximum(bx1, px1), 0.0)
    wy = jnp.maximum(jnp.minimum(by2, py2) - jnp.maximum(by1, py1), 0.0)
    inter = wx * wy
    return inter / (w * h + pw * ph - inter + 1e-10)


def _best_kernel(tgt_ref, pr_ref, best_ref):
    cx = tgt_ref[0, 0, 0]
    cy = tgt_ref[0, 0, 1]
    w = tgt_ref[0, 0, 2]
    h = tgt_ref[0, 0, 3]
    px, py, pw, ph = pr_ref[0], pr_ref[1], pr_ref[2], pr_ref[3]
    iou = _iou_terms(cx, cy, w, h, px, py, pw, ph)
    m = jnp.max(iou)
    r_i = lax.broadcasted_iota(jnp.int32, iou.shape, 0).astype(F32)
    c_i = lax.broadcasted_iota(jnp.int32, iou.shape, 1).astype(F32)
    gp = r_i * float(P8) + c_i
    best = jnp.min(jnp.where(iou == m, gp, F32(P)))
    best_ref[...] = jnp.full((1, 1, 128), best, F32)


def _main_kernel(tgt_ref, best_ref, pr_ref, sc_ref, lc_ref, wsse_ref,
                 didx_ref, wloc_ref, ce_ref, acc_ref, wslab_ref):
    n = pl.program_id(1)
    b = pl.program_id(0)
    cx = tgt_ref[0, 0, 0]
    cy = tgt_ref[0, 0, 1]
    w = tgt_ref[0, 0, 2]
    h = tgt_ref[0, 0, 3]
    sn = tgt_ref[0, 0, 5]
    cs = tgt_ref[0, 0, 6]
    lab_i = lax.convert_element_type(tgt_ref[0, 0, 7], jnp.int32)
    best = best_ref[0, 0, 0]
    px, py, pw, ph = (pr_ref[0, 0], pr_ref[1, 0], pr_ref[2, 0], pr_ref[3, 0])

    @pl.when(n == 0)
    def _():
        wslab_ref[...] = ((didx_ref[...] == 10 ** 6)
                          | (didx_ref[...] == lab_i)).astype(F32)

    iou = _iou_terms(cx, cy, w, h, px, py, pw, ph)
    g_i = lax.broadcasted_iota(jnp.int32, iou.shape, 0)
    l_i = lax.broadcasted_iota(jnp.int32, iou.shape, 1)
    gp = ((n * (GT * 128) + g_i * 128 + l_i)).astype(F32)
    pos = (iou >= THR) | (gp == best)
    posf = pos.astype(F32)
    npos_t = jnp.sum(posf)

    x = sc_ref[0]
    e = jnp.exp(x)
    sse = jnp.sum(lax.dot_general(e, wsse_ref[...], _BN,
                                  preferred_element_type=F32), axis=0)
    both = jnp.sum(lax.dot_general(x, wslab_ref[...], _BN,
                                   preferred_element_type=F32), axis=0)
    s0 = both[:, 0:128]
    slab = both[:, 128:256]
    lse = jnp.log(sse)
    ce0 = lse - s0
    conf_t = jnp.sum(posf * (lse - slab))
    ce_ref[0] = jnp.where(pos, 0.0, ce0)

    xl = lc_ref[0]
    lt = jnp.sum(lax.dot_general(xl, wloc_ref[...], _BN,
                                 preferred_element_type=F32), axis=0)
    d0 = lt[:, 0:128] - (cx - px) / (pw * 0.1)
    d1 = lt[:, 128:256] - (cy - py) / (ph * 0.1)
    d2 = lt[:, 256:384] - 5.0 * jnp.log(w / pw)
    d3 = lt[:, 384:512] - 5.0 * jnp.log(h / ph)
    d4 = lt[:, 512:640] - sn
    d5 = lt[:, 640:768] - cs
    loc_t = 0.0
    for d in (d0, d1, d2, d3):
        ad = jnp.abs(d)
        loc_t += jnp.sum(posf * jnp.where(ad < 1.0, 0.5 * d * d, ad - 0.5))
    ang_t = jnp.sum(posf * (d4 * d4 + d5 * d5))

    lane = lax.broadcasted_iota(jnp.int32, (1, 1, 128), 2)
    vec = (jnp.where(lane == 0, npos_t, 0.0)
           + jnp.where(lane == 1, conf_t, 0.0)
           + jnp.where(lane == 2, loc_t, 0.0)
           + jnp.where(lane == 3, ang_t, 0.0))

    @pl.when(n == 0)
    def _():
        acc_ref[...] = vec

    @pl.when(n != 0)
    def _():
        acc_ref[...] = acc_ref[...] + vec


def _mine_kernel(ce_ref, acc_ref, tot_ref):
    g = pl.program_id(0)
    x = ce_ref[...]
    bits = lax.bitcast_convert_type(x, jnp.int32)
    npos = acc_ref[:, :, 0:1]
    kf = jnp.minimum(npos * 3.0, F32(P))

    def body(_, lohi):
        lo, hi = lohi
        mid = lo + lax.div(hi - lo, jnp.int32(2))
        cnt = jnp.sum((bits > mid).astype(F32), axis=(1, 2), keepdims=True)
        take = cnt >= kf
        return (jnp.where(take, mid, lo), jnp.where(take, hi, mid))

    init = (jnp.full((IB, 1, 1), -1, jnp.int32),
            jnp.full((IB, 1, 1), 0x7F800000, jnp.int32))
    _, hi = lax.fori_loop(0, 31, body, init)
    vkf = lax.bitcast_convert_type(hi, F32)
    gtm = bits > hi
    sum_gt = jnp.sum(jnp.where(gtm, x, 0.0), axis=(1, 2), keepdims=True)
    cnt_gt = jnp.sum(gtm.astype(F32), axis=(1, 2), keepdims=True)
    topk = sum_gt + (kf - cnt_gt) * vkf

    lane = lax.broadcasted_iota(jnp.int32, (1, 128), 1)
    vec = (jnp.where(lane == 0, jnp.sum(npos), 0.0)
           + jnp.where(lane == 1, jnp.sum(acc_ref[:, :, 1:2] + topk), 0.0)
           + jnp.where(lane == 2, jnp.sum(acc_ref[:, :, 2:3]), 0.0)
           + jnp.where(lane == 3, jnp.sum(acc_ref[:, :, 3:4]), 0.0))

    @pl.when(g == 0)
    def _():
        tot_ref[...] = vec

    @pl.when(g != 0)
    def _():
        tot_ref[...] = tot_ref[...] + vec


def kernel(predicted_locs, predicted_scores, target, priors_cxcy):
    prt = priors_cxcy.T
    priors_b = prt.reshape(4, 8, P8)
    priors_m = prt.reshape(4, NT, GT, 128)
    scf = predicted_scores.reshape(B, NT * GT, C, 128)
    lcf = predicted_locs.reshape(B, NT * GT, 6, 128)
    wsse = jnp.asarray(W_SSE)
    didx = jnp.asarray(D_CMB)
    wloc = jnp.asarray(W_LOC)

    best = pl.pallas_call(
        _best_kernel,
        grid=(B,),
        in_specs=[
            pl.BlockSpec((1, 1, 8), lambda b: (b, 0, 0)),
            pl.BlockSpec((4, 8, P8), lambda b: (0, 0, 0)),
        ],
        out_specs=pl.BlockSpec((1, 1, 128), lambda b: (b, 0, 0)),
        out_shape=jax.ShapeDtypeStruct((B, 1, 128), F32),
    )(target, priors_b)

    ce_neg, acc = pl.pallas_call(
        _main_kernel,
        grid=(B, NT),
        in_specs=[
            pl.BlockSpec((1, 1, 8), lambda b, n: (b, 0, 0)),
            pl.BlockSpec((1, 1, 128), lambda b, n: (b, 0, 0)),
            pl.BlockSpec((4, 1, GT, 128), lambda b, n: (0, n, 0, 0)),
            pl.BlockSpec((1, GT, C, 128), lambda b, n: (b, n, 0, 0)),
            pl.BlockSpec((1, GT, 6, 128), lambda b, n: (b, n, 0, 0)),
            pl.BlockSpec((C, 128, 128), lambda b, n: (0, 0, 0)),
            pl.BlockSpec((C, 128, 256), lambda b, n: (0, 0, 0)),
            pl.BlockSpec((6, 128, 768), lambda b, n: (0, 0, 0)),
        ],
        out_specs=[
            pl.BlockSpec((1, NT * GT, 128), lambda b, n: (b, 0, 0)),
            pl.BlockSpec((1, 1, 128), lambda b, n: (b, 0, 0)),
        ],
        out_shape=[
            jax.ShapeDtypeStruct((B, NT * GT, 128), F32),
            jax.ShapeDtypeStruct((B, 1, 128), F32),
        ],
        scratch_shapes=[pltpu.VMEM((C, 128, 256), F32)],
    )(target, best, priors_m, scf, lcf, wsse, didx, wloc)

    tot = pl.pallas_call(
        _mine_kernel,
        grid=(B // IB,),
        in_specs=[
            pl.BlockSpec((IB, NT * GT, 128), lambda g: (g, 0, 0)),
            pl.BlockSpec((IB, 1, 128), lambda g: (g, 0, 0)),
        ],
        out_specs=pl.BlockSpec((1, 128), lambda g: (0, 0)),
        out_shape=jax.ShapeDtypeStruct((1, 128), F32),
    )(ce_neg, acc)

    n = tot[0, 0]
    conf = tot[0, 1] / n
    loc = tot[0, 2] / (n * 4.0)
    ang = 25.0 * tot[0, 3] / (n * 2.0)
    return (conf, loc, ang, conf + loc + ang)
